# baseline (speedup 1.0000x reference)
import jax
import jax.numpy as jnp
from jax.experimental import pallas as pl
from jax.experimental.pallas import tpu as pltpu

_D = 4096
_N = 64
_LO = (_D - _N) // 2
_BLK = 256
_NBLK = _D // _BLK


def _mlp_argmax_kernel(x_ref, w1a_ref, w1b_ref, b1_ref, b2_ref, wm_ref,
                       w2_ref, out_ref, h_ref, t2_ref, m_ref):
    j = pl.program_id(0)

    @pl.when(j == 0)
    def _init():
        w1w = jnp.concatenate([w1a_ref[...], w1b_ref[...]], axis=0)
        xw = x_ref[:, _LO:_LO + _N]
        pre = jnp.dot(xw, w1w, preferred_element_type=jnp.float32)
        h_ref[...] = jnp.maximum(pre + b1_ref[...] + w1w, 0.0)
        b2 = jnp.broadcast_to(b2_ref[...], (_N, _D))
        t2_ref[...] = b2
        m_ref[...] = jnp.sum(b2 * wm_ref[...], axis=1, keepdims=True)

    h_blk = h_ref[:, pl.ds(j * _BLK, _BLK)]
    delta = jnp.dot(h_blk, w2_ref[...], preferred_element_type=jnp.float32)
    t2_ref[...] += delta
    m_ref[...] += jnp.sum(delta * wm_ref[...], axis=1, keepdims=True)

    @pl.when(j == _NBLK - 1)
    def _fin():
        metric = m_ref[...]
        mmax = jnp.max(metric)
        iota = jax.lax.broadcasted_iota(jnp.int32, (_N, 1), 0)
        idx = jnp.min(jnp.where(metric == mmax, iota, _N))
        out_ref[...] = t2_ref[pl.ds(idx, 1), :]


@jax.jit
def kernel(x, W1, b1, W2, b2, w_metric):
    xr = x.reshape(1, _D)
    b1r = b1.reshape(1, _D)
    b2r = b2.reshape(1, _D)
    wmr = w_metric.reshape(1, _D)

    out = pl.pallas_call(
        _mlp_argmax_kernel,
        grid=(_NBLK,),
        in_specs=[
            pl.BlockSpec((1, _D), lambda j: (0, 0)),
            pl.BlockSpec((32, _D), lambda j: (_LO // 32, 0)),
            pl.BlockSpec((32, _D), lambda j: (_LO // 32 + 1, 0)),
            pl.BlockSpec((1, _D), lambda j: (0, 0)),
            pl.BlockSpec((1, _D), lambda j: (0, 0)),
            pl.BlockSpec((1, _D), lambda j: (0, 0)),
            pl.BlockSpec((_BLK, _D), lambda j: (j, 0)),
        ],
        out_specs=pl.BlockSpec((1, _D), lambda j: (0, 0)),
        out_shape=jax.ShapeDtypeStruct((1, _D), jnp.float32),
        scratch_shapes=[
            pltpu.VMEM((_N, _D), jnp.float32),
            pltpu.VMEM((_N, _D), jnp.float32),
            pltpu.VMEM((_N, 1), jnp.float32),
        ],
        compiler_params=pltpu.CompilerParams(
            dimension_semantics=("arbitrary",),
        ),
    )(xr, W1, W1, b1r, b2r, wmr, W2)
    return out.reshape(_D)


# speedup vs baseline: 1.1403x; 1.1403x over previous
import jax
import jax.numpy as jnp
from jax.experimental import pallas as pl
from jax.experimental.pallas import tpu as pltpu

_D = 4096
_N = 64
_LO = (_D - _N) // 2
_BLK = 512
_NBLK = _D // _BLK


def _mlp_argmax_kernel(x_ref, w1a_ref, w1b_ref, b1_ref, b2_ref, wm_ref,
                       w2_ref, out_ref, h_ref, t2_ref):
    j = pl.program_id(0)

    @pl.when(j == 0)
    def _init():
        w1w = jnp.concatenate([w1a_ref[...], w1b_ref[...]], axis=0)
        xw = x_ref[:, _LO:_LO + _N]
        pre = jnp.dot(xw, w1w, preferred_element_type=jnp.float32)
        h_ref[...] = jnp.maximum(pre + b1_ref[...] + w1w, 0.0)
        t2_ref[...] = jnp.broadcast_to(b2_ref[...], (_N, _D))

    h_blk = h_ref[:, pl.ds(j * _BLK, _BLK)]
    t2_ref[...] += jnp.dot(h_blk, w2_ref[...],
                           preferred_element_type=jnp.float32)

    @pl.when(j == _NBLK - 1)
    def _fin():
        metric = jnp.sum(t2_ref[...] * wm_ref[...], axis=1,
                         keepdims=True)
        mmax = jnp.max(metric)
        iota = jax.lax.broadcasted_iota(jnp.int32, (_N, 1), 0)
        idx = jnp.min(jnp.where(metric == mmax, iota, _N))
        out_ref[...] = t2_ref[pl.ds(idx, 1), :]


@jax.jit
def kernel(x, W1, b1, W2, b2, w_metric):
    xr = x.reshape(1, _D)
    b1r = b1.reshape(1, _D)
    b2r = b2.reshape(1, _D)
    wmr = w_metric.reshape(1, _D)

    out = pl.pallas_call(
        _mlp_argmax_kernel,
        grid=(_NBLK,),
        in_specs=[
            pl.BlockSpec((1, _D), lambda j: (0, 0)),
            pl.BlockSpec((32, _D), lambda j: (_LO // 32, 0)),
            pl.BlockSpec((32, _D), lambda j: (_LO // 32 + 1, 0)),
            pl.BlockSpec((1, _D), lambda j: (0, 0)),
            pl.BlockSpec((1, _D), lambda j: (0, 0)),
            pl.BlockSpec((1, _D), lambda j: (0, 0)),
            pl.BlockSpec((_BLK, _D), lambda j: (j, 0)),
        ],
        out_specs=pl.BlockSpec((1, _D), lambda j: (0, 0)),
        out_shape=jax.ShapeDtypeStruct((1, _D), jnp.float32),
        scratch_shapes=[
            pltpu.VMEM((_N, _D), jnp.float32),
            pltpu.VMEM((_N, _D), jnp.float32),
        ],
        compiler_params=pltpu.CompilerParams(
            dimension_semantics=("arbitrary",),
        ),
    )(xr, W1, W1, b1r, b2r, wmr, W2)
    return out.reshape(_D)


# speedup vs baseline: 1.1475x; 1.0063x over previous
import jax
import jax.numpy as jnp
from jax.experimental import pallas as pl
from jax.experimental.pallas import tpu as pltpu

_D = 4096
_N = 64
_LO = (_D - _N) // 2
_BLK = 512
_NBLK = _D // _BLK


def _mlp_argmax_kernel(x_ref, w1a_ref, w1b_ref, b1_ref, b2_ref, wm_ref,
                       w2_ref, out_ref, h_ref, t2_ref, m_ref):
    j = pl.program_id(0)

    @pl.when(j == 0)
    def _init():
        w1w = jnp.concatenate([w1a_ref[...], w1b_ref[...]], axis=0)
        xw = x_ref[:, _LO:_LO + _N]
        pre = jnp.dot(xw, w1w, preferred_element_type=jnp.float32)
        h_ref[...] = jnp.maximum(pre + b1_ref[...] + w1w, 0.0)
        b2 = jnp.broadcast_to(b2_ref[...], (_N, _D))
        t2_ref[...] = b2
        m_ref[...] = jnp.sum(b2 * wm_ref[...], axis=1, keepdims=True)

    h_blk = h_ref[:, pl.ds(j * _BLK, _BLK)]
    delta = jnp.dot(h_blk, w2_ref[...], preferred_element_type=jnp.float32)
    t2_ref[...] += delta
    m_ref[...] += jnp.sum(delta * wm_ref[...], axis=1, keepdims=True)

    @pl.when(j == _NBLK - 1)
    def _fin():
        metric = m_ref[...]
        mmax = jnp.max(metric)
        iota = jax.lax.broadcasted_iota(jnp.int32, (_N, 1), 0)
        idx = jnp.min(jnp.where(metric == mmax, iota, _N))
        out_ref[...] = t2_ref[pl.ds(idx, 1), :]


@jax.jit
def kernel(x, W1, b1, W2, b2, w_metric):
    xr = x.reshape(1, _D)
    b1r = b1.reshape(1, _D)
    b2r = b2.reshape(1, _D)
    wmr = w_metric.reshape(1, _D)

    out = pl.pallas_call(
        _mlp_argmax_kernel,
        grid=(_NBLK,),
        in_specs=[
            pl.BlockSpec((1, _D), lambda j: (0, 0)),
            pl.BlockSpec((32, _D), lambda j: (_LO // 32, 0)),
            pl.BlockSpec((32, _D), lambda j: (_LO // 32 + 1, 0)),
            pl.BlockSpec((1, _D), lambda j: (0, 0)),
            pl.BlockSpec((1, _D), lambda j: (0, 0)),
            pl.BlockSpec((1, _D), lambda j: (0, 0)),
            pl.BlockSpec((_BLK, _D), lambda j: (j, 0)),
        ],
        out_specs=pl.BlockSpec((1, _D), lambda j: (0, 0)),
        out_shape=jax.ShapeDtypeStruct((1, _D), jnp.float32),
        scratch_shapes=[
            pltpu.VMEM((_N, _D), jnp.float32),
            pltpu.VMEM((_N, _D), jnp.float32),
            pltpu.VMEM((_N, 1), jnp.float32),
        ],
        compiler_params=pltpu.CompilerParams(
            dimension_semantics=("arbitrary",),
        ),
    )(xr, W1, W1, b1r, b2r, wmr, W2)
    return out.reshape(_D)
